# baseline (speedup 1.0000x reference)
import functools

import numpy as np
import jax
import jax.numpy as jnp
from jax import lax
from jax.experimental import pallas as pl
from jax.experimental.pallas import tpu as pltpu
from jax.experimental.pallas import tpu_sc as plsc

_B = 4
_N = 4096
_K = 16
_L = 16
_NBLK = _N // _L
_NC = 2
_NS = 16
_NW = _NC * _NS
_WPB = _NW // _B
_ROWS = _N // _WPB
_BIG = np.float32(3.0e38)
_SAMPLE = 256
_HCAP = _N + _L


def _sqrt16(a):
    i = plsc.bitcast(a, jnp.int32)
    i = jnp.int32(0x5F3759DF) - (i >> 1)
    y = plsc.bitcast(i, jnp.float32)
    ah = a * jnp.float32(0.5)
    y = y * (jnp.float32(1.5) - ah * y * y)
    y = y * (jnp.float32(1.5) - ah * y * y)
    y = y * (jnp.float32(1.5) - ah * y * y)
    return jnp.where(a > 0.0, a * y, jnp.float32(0.0))


def _body(rx_hbm, ry_hbm, rz_hbm, px_hbm, py_hbm, pz_hbm, out_hbm,
          xs, ys, zs, pxs, pys, pzs, hitk, hitv, accv):
    wid = lax.axis_index("s") * _NC + lax.axis_index("c")
    batch = wid // _WPB
    row0 = (wid % _WPB) * _ROWS

    boff = batch * _N
    pltpu.sync_copy(rx_hbm.at[pl.ds(boff, _N)], xs)
    pltpu.sync_copy(ry_hbm.at[pl.ds(boff, _N)], ys)
    pltpu.sync_copy(rz_hbm.at[pl.ds(boff, _N)], zs)
    pltpu.sync_copy(px_hbm.at[pl.ds(boff, _N)], pxs)
    pltpu.sync_copy(py_hbm.at[pl.ds(boff, _N)], pys)
    pltpu.sync_copy(pz_hbm.at[pl.ds(boff, _N)], pzs)

    iota = lax.iota(jnp.int32, _L)

    def row_body(i, acc_comp):
        acc, comp = acc_comp
        r = row0 + i
        rv = jnp.full((_L,), r, jnp.int32)
        qx = plsc.load_gather(xs, [rv])
        qy = plsc.load_gather(ys, [rv])
        qz = plsc.load_gather(zs, [rv])

        def dist_block(base):
            xv = xs[pl.ds(base, _L)]
            yv = ys[pl.ds(base, _L)]
            zv = zs[pl.ds(base, _L)]
            dx = xv - qx
            dy = yv - qy
            dz = zv - qz
            d2 = dx * dx + dy * dy + dz * dz
            idxv = iota + base
            return jnp.where(idxv == rv, _BIG, d2), idxv

        def merge(carry, d2, idxv):
            bk, bv = carry
            sk, sv = plsc.sort_key_val(d2, idxv)
            rk = lax.rev(sk, (0,))
            rsv = lax.rev(sv, (0,))
            take = bk <= rk
            mk = jnp.where(take, bk, rk)
            mv = jnp.where(take, bv, rsv)
            nk, nv = plsc.sort_key_val(mk, mv)
            return nk, nv

        def p1_body(c, carry):
            d2, idxv = dist_block(c * _L)
            return merge(carry, d2, idxv)

        bk0 = jnp.full((_L,), _BIG, jnp.float32)
        bv0 = jnp.zeros((_L,), jnp.int32)
        bk, bv = lax.fori_loop(0, _SAMPLE // _L, p1_body, (bk0, bv0))

        t = jnp.max(bk)

        def p2_body(c, cnt):
            d2, idxv = dist_block(c * _L)
            hit = d2 < t
            plsc.store_compressed(hitk.at[pl.ds(cnt, _L)], d2, mask=hit)
            plsc.store_compressed(hitv.at[pl.ds(cnt, _L)], idxv, mask=hit)
            return cnt + jnp.sum(hit.astype(jnp.int32))

        cnt = lax.fori_loop(_SAMPLE // _L, _NBLK, p2_body, jnp.int32(0))

        def p3_body(j, carry):
            base = j * _L
            hk = hitk[pl.ds(base, _L)]
            hv = hitv[pl.ds(base, _L)]
            hk = jnp.where(iota + base < cnt, hk, _BIG)
            return merge(carry, hk, hv)

        nit = (cnt + _L - 1) // _L
        bk, bv = lax.fori_loop(0, nit, p3_body, (bk, bv))

        dist_ref = _sqrt16(bk)

        qpx = plsc.load_gather(pxs, [rv])
        qpy = plsc.load_gather(pys, [rv])
        qpz = plsc.load_gather(pzs, [rv])
        nx = plsc.load_gather(pxs, [bv])
        ny = plsc.load_gather(pys, [bv])
        nz = plsc.load_gather(pzs, [bv])
        ddx = nx - qpx
        ddy = ny - qpy
        ddz = nz - qpz
        dist = _sqrt16(ddx * ddx + ddy * ddy + ddz * ddz)
        y = jnp.abs(dist_ref - dist) - comp
        t = acc + y
        comp = (t - acc) - y
        return t, comp

    zero = jnp.zeros((_L,), jnp.float32)
    acc, _ = lax.fori_loop(0, _ROWS, row_body, (zero, zero))
    accv[...] = acc
    pltpu.sync_copy(accv, out_hbm.at[wid])


@jax.jit
def _partials(rx, ry, rz, px, py, pz):
    mesh = plsc.VectorSubcoreMesh(
        core_axis_name="c", subcore_axis_name="s",
        num_cores=_NC, num_subcores=_NS)
    f = pl.kernel(
        _body,
        out_type=jax.ShapeDtypeStruct((_NW, _L), jnp.float32),
        mesh=mesh,
        scratch_types=[
            pltpu.VMEM((_N,), jnp.float32),
            pltpu.VMEM((_N,), jnp.float32),
            pltpu.VMEM((_N,), jnp.float32),
            pltpu.VMEM((_N,), jnp.float32),
            pltpu.VMEM((_N,), jnp.float32),
            pltpu.VMEM((_N,), jnp.float32),
            pltpu.VMEM((_HCAP,), jnp.float32),
            pltpu.VMEM((_HCAP,), jnp.int32),
            pltpu.VMEM((_L,), jnp.float32),
        ],
        compiler_params=pltpu.CompilerParams(needs_layout_passes=False),
    )
    return f(rx, ry, rz, px, py, pz)


def kernel(points_ref, points):
    rx, ry, rz = (points_ref[:, :, i].reshape(-1) for i in range(3))
    px, py, pz = (points[:, :, i].reshape(-1) for i in range(3))
    partials = _partials(rx, ry, rz, px, py, pz)
    return jnp.sum(partials) / jnp.float32(_B * _N * _K)


# speedup vs baseline: 1.0011x; 1.0011x over previous
import functools

import numpy as np
import jax
import jax.numpy as jnp
from jax import lax
from jax.experimental import pallas as pl
from jax.experimental.pallas import tpu as pltpu
from jax.experimental.pallas import tpu_sc as plsc

_B = 4
_N = 4096
_K = 16
_L = 16
_NBLK = _N // _L
_NC = 2
_NS = 16
_NW = _NC * _NS
_WPB = _NW // _B
_ROWS = _N // _WPB
_BIG = np.float32(3.0e38)
_SAMPLE = 256
_HCAP = _N + _L


def _sqrt16(a):
    i = plsc.bitcast(a, jnp.int32)
    i = jnp.int32(0x5F3759DF) - (i >> 1)
    y = plsc.bitcast(i, jnp.float32)
    ah = a * jnp.float32(0.5)
    y = y * (jnp.float32(1.5) - ah * y * y)
    y = y * (jnp.float32(1.5) - ah * y * y)
    y = y * (jnp.float32(1.5) - ah * y * y)
    return jnp.where(a > 0.0, a * y, jnp.float32(0.0))


def _body(rx_hbm, ry_hbm, rz_hbm, px_hbm, py_hbm, pz_hbm, out_hbm,
          xs, ys, zs, pxs, pys, pzs, hitk, hitv, accv):
    wid = lax.axis_index("s") * _NC + lax.axis_index("c")
    batch = wid // _WPB
    row0 = (wid % _WPB) * _ROWS

    boff = batch * _N
    pltpu.sync_copy(rx_hbm.at[pl.ds(boff, _N)], xs)
    pltpu.sync_copy(ry_hbm.at[pl.ds(boff, _N)], ys)
    pltpu.sync_copy(rz_hbm.at[pl.ds(boff, _N)], zs)
    pltpu.sync_copy(px_hbm.at[pl.ds(boff, _N)], pxs)
    pltpu.sync_copy(py_hbm.at[pl.ds(boff, _N)], pys)
    pltpu.sync_copy(pz_hbm.at[pl.ds(boff, _N)], pzs)

    iota = lax.iota(jnp.int32, _L)

    def row_body(i, acc_comp):
        acc, comp = acc_comp
        r = row0 + i
        rv = jnp.full((_L,), r, jnp.int32)
        qx = plsc.load_gather(xs, [rv])
        qy = plsc.load_gather(ys, [rv])
        qz = plsc.load_gather(zs, [rv])

        def dist_block(base):
            xv = xs[pl.ds(base, _L)]
            yv = ys[pl.ds(base, _L)]
            zv = zs[pl.ds(base, _L)]
            dx = xv - qx
            dy = yv - qy
            dz = zv - qz
            d2 = dx * dx + dy * dy + dz * dz
            idxv = iota + base
            return jnp.where(idxv == rv, _BIG, d2), idxv

        def merge(carry, d2, idxv):
            bk, bv = carry
            sk, sv = plsc.sort_key_val(d2, idxv)
            rk = lax.rev(sk, (0,))
            rsv = lax.rev(sv, (0,))
            take = bk <= rk
            mk = jnp.where(take, bk, rk)
            mv = jnp.where(take, bv, rsv)
            nk, nv = plsc.sort_key_val(mk, mv)
            return nk, nv

        def p1_body(c, carry):
            d2, idxv = dist_block(c * _L)
            return merge(carry, d2, idxv)

        bk0 = jnp.full((_L,), _BIG, jnp.float32)
        bv0 = jnp.zeros((_L,), jnp.int32)
        bk, bv = lax.fori_loop(0, _SAMPLE // _L, p1_body, (bk0, bv0))

        t = jnp.max(bk)

        def p2_body(c, cnt):
            d2, idxv = dist_block(c * _L)
            hit = d2 < t
            plsc.store_compressed(hitk.at[pl.ds(cnt, _L)], d2, mask=hit)
            plsc.store_compressed(hitv.at[pl.ds(cnt, _L)], idxv, mask=hit)
            return cnt + plsc.all_reduce_population_count(hit)[0]

        cnt = lax.fori_loop(_SAMPLE // _L, _NBLK, p2_body, jnp.int32(0))

        def p3_body(j, carry):
            base = j * _L
            hk = hitk[pl.ds(base, _L)]
            hv = hitv[pl.ds(base, _L)]
            hk = jnp.where(iota + base < cnt, hk, _BIG)
            return merge(carry, hk, hv)

        nit = (cnt + _L - 1) // _L
        bk, bv = lax.fori_loop(0, nit, p3_body, (bk, bv))

        dist_ref = _sqrt16(bk)

        qpx = plsc.load_gather(pxs, [rv])
        qpy = plsc.load_gather(pys, [rv])
        qpz = plsc.load_gather(pzs, [rv])
        nx = plsc.load_gather(pxs, [bv])
        ny = plsc.load_gather(pys, [bv])
        nz = plsc.load_gather(pzs, [bv])
        ddx = nx - qpx
        ddy = ny - qpy
        ddz = nz - qpz
        dist = _sqrt16(ddx * ddx + ddy * ddy + ddz * ddz)
        y = jnp.abs(dist_ref - dist) - comp
        t = acc + y
        comp = (t - acc) - y
        return t, comp

    zero = jnp.zeros((_L,), jnp.float32)
    acc, _ = lax.fori_loop(0, _ROWS, row_body, (zero, zero))
    accv[...] = acc
    pltpu.sync_copy(accv, out_hbm.at[wid])


@jax.jit
def _partials(rx, ry, rz, px, py, pz):
    mesh = plsc.VectorSubcoreMesh(
        core_axis_name="c", subcore_axis_name="s",
        num_cores=_NC, num_subcores=_NS)
    f = pl.kernel(
        _body,
        out_type=jax.ShapeDtypeStruct((_NW, _L), jnp.float32),
        mesh=mesh,
        scratch_types=[
            pltpu.VMEM((_N,), jnp.float32),
            pltpu.VMEM((_N,), jnp.float32),
            pltpu.VMEM((_N,), jnp.float32),
            pltpu.VMEM((_N,), jnp.float32),
            pltpu.VMEM((_N,), jnp.float32),
            pltpu.VMEM((_N,), jnp.float32),
            pltpu.VMEM((_HCAP,), jnp.float32),
            pltpu.VMEM((_HCAP,), jnp.int32),
            pltpu.VMEM((_L,), jnp.float32),
        ],
        compiler_params=pltpu.CompilerParams(needs_layout_passes=False),
    )
    return f(rx, ry, rz, px, py, pz)


def kernel(points_ref, points):
    rx, ry, rz = (points_ref[:, :, i].reshape(-1) for i in range(3))
    px, py, pz = (points[:, :, i].reshape(-1) for i in range(3))
    partials = _partials(rx, ry, rz, px, py, pz)
    return jnp.sum(partials) / jnp.float32(_B * _N * _K)


# speedup vs baseline: 3.2705x; 3.2669x over previous
import functools

import numpy as np
import jax
import jax.numpy as jnp
from jax import lax
from jax.experimental import pallas as pl
from jax.experimental.pallas import tpu as pltpu
from jax.experimental.pallas import tpu_sc as plsc

_B = 4
_N = 4096
_K = 16
_L = 16
_NBLK = _N // _L
_NC = 2
_NS = 16
_NW = _NC * _NS
_WPB = _NW // _B
_ROWS = _N // _WPB
_BIG = np.float32(3.0e38)
_SAMPLE = 256
_HCAP = _N + _L


def _sqrt16(a):
    i = plsc.bitcast(a, jnp.int32)
    i = jnp.int32(0x5F3759DF) - (i >> 1)
    y = plsc.bitcast(i, jnp.float32)
    ah = a * jnp.float32(0.5)
    y = y * (jnp.float32(1.5) - ah * y * y)
    y = y * (jnp.float32(1.5) - ah * y * y)
    y = y * (jnp.float32(1.5) - ah * y * y)
    return jnp.where(a > 0.0, a * y, jnp.float32(0.0))


def _body(rx_hbm, ry_hbm, rz_hbm, px_hbm, py_hbm, pz_hbm, out_hbm,
          xs, ys, zs, pxs, pys, pzs, hitk, hitv, accv):
    wid = lax.axis_index("s") * _NC + lax.axis_index("c")
    batch = wid // _WPB
    row0 = (wid % _WPB) * _ROWS

    boff = batch * _N
    pltpu.sync_copy(rx_hbm.at[pl.ds(boff, _N)], xs)
    pltpu.sync_copy(ry_hbm.at[pl.ds(boff, _N)], ys)
    pltpu.sync_copy(rz_hbm.at[pl.ds(boff, _N)], zs)
    pltpu.sync_copy(px_hbm.at[pl.ds(boff, _N)], pxs)
    pltpu.sync_copy(py_hbm.at[pl.ds(boff, _N)], pys)
    pltpu.sync_copy(pz_hbm.at[pl.ds(boff, _N)], pzs)

    iota = lax.iota(jnp.int32, _L)

    def row_body(i, acc_comp):
        acc, comp = acc_comp
        r = row0 + i
        rv = jnp.full((_L,), r, jnp.int32)
        qx = plsc.load_gather(xs, [rv])
        qy = plsc.load_gather(ys, [rv])
        qz = plsc.load_gather(zs, [rv])

        def dist_block(base):
            xv = xs[pl.ds(base, _L)]
            yv = ys[pl.ds(base, _L)]
            zv = zs[pl.ds(base, _L)]
            dx = xv - qx
            dy = yv - qy
            dz = zv - qz
            d2 = dx * dx + dy * dy + dz * dz
            idxv = iota + base
            return jnp.where(idxv == rv, _BIG, d2), idxv

        def merge(carry, d2, idxv):
            bk, bv = carry
            sk, sv = plsc.sort_key_val(d2, idxv)
            rk = lax.rev(sk, (0,))
            rsv = lax.rev(sv, (0,))
            take = bk <= rk
            mk = jnp.where(take, bk, rk)
            mv = jnp.where(take, bv, rsv)
            nk, nv = plsc.sort_key_val(mk, mv)
            return nk, nv

        def p1_body(c, carry):
            d2, idxv = dist_block(c * _L)
            return merge(carry, d2, idxv)

        bk0 = jnp.full((_L,), _BIG, jnp.float32)
        bv0 = jnp.zeros((_L,), jnp.int32)
        bk, bv = lax.fori_loop(0, _SAMPLE // _L, p1_body, (bk0, bv0))

        t = jnp.max(bk)

        @plsc.parallel_loop(_SAMPLE // _L, _NBLK, unroll=4, carry=jnp.int32(0))
        def p2_cnt(c, cnt):
            d2, idxv = dist_block(c * _L)
            hit = d2 < t
            plsc.store_compressed(hitk.at[pl.ds(cnt, _L)], d2, mask=hit)
            plsc.store_compressed(hitv.at[pl.ds(cnt, _L)], idxv, mask=hit)
            return cnt + plsc.all_reduce_population_count(hit)[0]

        cnt = p2_cnt

        def p3_body(j, carry):
            base = j * _L
            hk = hitk[pl.ds(base, _L)]
            hv = hitv[pl.ds(base, _L)]
            hk = jnp.where(iota + base < cnt, hk, _BIG)
            return merge(carry, hk, hv)

        nit = (cnt + _L - 1) // _L
        bk, bv = lax.fori_loop(0, nit, p3_body, (bk, bv))

        dist_ref = _sqrt16(bk)

        qpx = plsc.load_gather(pxs, [rv])
        qpy = plsc.load_gather(pys, [rv])
        qpz = plsc.load_gather(pzs, [rv])
        nx = plsc.load_gather(pxs, [bv])
        ny = plsc.load_gather(pys, [bv])
        nz = plsc.load_gather(pzs, [bv])
        ddx = nx - qpx
        ddy = ny - qpy
        ddz = nz - qpz
        dist = _sqrt16(ddx * ddx + ddy * ddy + ddz * ddz)
        y = jnp.abs(dist_ref - dist) - comp
        t = acc + y
        comp = (t - acc) - y
        return t, comp

    zero = jnp.zeros((_L,), jnp.float32)
    acc, _ = lax.fori_loop(0, _ROWS, row_body, (zero, zero))
    accv[...] = acc
    pltpu.sync_copy(accv, out_hbm.at[wid])


@jax.jit
def _partials(rx, ry, rz, px, py, pz):
    mesh = plsc.VectorSubcoreMesh(
        core_axis_name="c", subcore_axis_name="s",
        num_cores=_NC, num_subcores=_NS)
    f = pl.kernel(
        _body,
        out_type=jax.ShapeDtypeStruct((_NW, _L), jnp.float32),
        mesh=mesh,
        scratch_types=[
            pltpu.VMEM((_N,), jnp.float32),
            pltpu.VMEM((_N,), jnp.float32),
            pltpu.VMEM((_N,), jnp.float32),
            pltpu.VMEM((_N,), jnp.float32),
            pltpu.VMEM((_N,), jnp.float32),
            pltpu.VMEM((_N,), jnp.float32),
            pltpu.VMEM((_HCAP,), jnp.float32),
            pltpu.VMEM((_HCAP,), jnp.int32),
            pltpu.VMEM((_L,), jnp.float32),
        ],
        compiler_params=pltpu.CompilerParams(needs_layout_passes=False),
    )
    return f(rx, ry, rz, px, py, pz)


def kernel(points_ref, points):
    rx, ry, rz = (points_ref[:, :, i].reshape(-1) for i in range(3))
    px, py, pz = (points[:, :, i].reshape(-1) for i in range(3))
    partials = _partials(rx, ry, rz, px, py, pz)
    return jnp.sum(partials) / jnp.float32(_B * _N * _K)


# speedup vs baseline: 3.3463x; 1.0232x over previous
import functools

import numpy as np
import jax
import jax.numpy as jnp
from jax import lax
from jax.experimental import pallas as pl
from jax.experimental.pallas import tpu as pltpu
from jax.experimental.pallas import tpu_sc as plsc

_B = 4
_N = 4096
_K = 16
_L = 16
_NBLK = _N // _L
_NC = 2
_NS = 16
_NW = _NC * _NS
_WPB = _NW // _B
_ROWS = _N // _WPB
_BIG = np.float32(3.0e38)
_SAMPLE = 256
_HCAP = _N + _L


def _sqrt16(a):
    i = plsc.bitcast(a, jnp.int32)
    i = jnp.int32(0x5F3759DF) - (i >> 1)
    y = plsc.bitcast(i, jnp.float32)
    ah = a * jnp.float32(0.5)
    y = y * (jnp.float32(1.5) - ah * y * y)
    y = y * (jnp.float32(1.5) - ah * y * y)
    y = y * (jnp.float32(1.5) - ah * y * y)
    return jnp.where(a > 0.0, a * y, jnp.float32(0.0))


def _body(rx_hbm, ry_hbm, rz_hbm, px_hbm, py_hbm, pz_hbm, out_hbm,
          xs, ys, zs, pxs, pys, pzs, hitk, hitv, hitk2, hitv2, accv):
    wid = lax.axis_index("s") * _NC + lax.axis_index("c")
    batch = wid // _WPB
    row0 = (wid % _WPB) * _ROWS

    boff = batch * _N
    pltpu.sync_copy(rx_hbm.at[pl.ds(boff, _N)], xs)
    pltpu.sync_copy(ry_hbm.at[pl.ds(boff, _N)], ys)
    pltpu.sync_copy(rz_hbm.at[pl.ds(boff, _N)], zs)
    pltpu.sync_copy(px_hbm.at[pl.ds(boff, _N)], pxs)
    pltpu.sync_copy(py_hbm.at[pl.ds(boff, _N)], pys)
    pltpu.sync_copy(pz_hbm.at[pl.ds(boff, _N)], pzs)

    iota = lax.iota(jnp.int32, _L)

    def d2_from(xv, yv, zv, q):
        dx = xv - q[0]
        dy = yv - q[1]
        dz = zv - q[2]
        return dx * dx + dy * dy + dz * dz

    def merge(carry, d2, idxv):
        bk, bv = carry
        sk, sv = plsc.sort_key_val(d2, idxv)
        rk = lax.rev(sk, (0,))
        rsv = lax.rev(sv, (0,))
        take = bk <= rk
        mk = jnp.where(take, bk, rk)
        mv = jnp.where(take, bv, rsv)
        nk, nv = plsc.sort_key_val(mk, mv)
        return nk, nv

    def sample_top16(q, rv):
        def p1_body(c, carry):
            base = c * _L
            xv = xs[pl.ds(base, _L)]
            yv = ys[pl.ds(base, _L)]
            zv = zs[pl.ds(base, _L)]
            d2 = d2_from(xv, yv, zv, q)
            idxv = iota + base
            d2 = jnp.where(idxv == rv, _BIG, d2)
            return merge(carry, d2, idxv)

        bk0 = jnp.full((_L,), _BIG, jnp.float32)
        bv0 = jnp.zeros((_L,), jnp.int32)
        return lax.fori_loop(0, _SAMPLE // _L, p1_body, (bk0, bv0))

    def merge_hits(cnt, bk, bv, hk_ref, hv_ref):
        def p3_body(j, carry):
            base = j * _L
            hk = hk_ref[pl.ds(base, _L)]
            hv = hv_ref[pl.ds(base, _L)]
            hk = jnp.where(iota + base < cnt, hk, _BIG)
            return merge(carry, hk, hv)

        nit = (cnt + _L - 1) // _L
        return lax.fori_loop(0, nit, p3_body, (bk, bv))

    def edge_loss(rv, bk, bv):
        dist_ref = _sqrt16(bk)
        qpx = plsc.load_gather(pxs, [rv])
        qpy = plsc.load_gather(pys, [rv])
        qpz = plsc.load_gather(pzs, [rv])
        nx = plsc.load_gather(pxs, [bv])
        ny = plsc.load_gather(pys, [bv])
        nz = plsc.load_gather(pzs, [bv])
        ddx = nx - qpx
        ddy = ny - qpy
        ddz = nz - qpz
        dist = _sqrt16(ddx * ddx + ddy * ddy + ddz * ddz)
        return jnp.abs(dist_ref - dist)

    def pair_body(p, acc_comp):
        acc, comp = acc_comp
        ra = row0 + 2 * p
        rva = jnp.full((_L,), ra, jnp.int32)
        rvb = rva + 1
        qa = tuple(plsc.load_gather(s, [rva]) for s in (xs, ys, zs))
        qb = tuple(plsc.load_gather(s, [rvb]) for s in (xs, ys, zs))

        bka, bva = sample_top16(qa, rva)
        bkb, bvb = sample_top16(qb, rvb)
        ta = jnp.max(bka)
        tb = jnp.max(bkb)

        zero2 = (jnp.int32(0), jnp.int32(0))

        @plsc.parallel_loop(_SAMPLE // _L, _NBLK, unroll=4, carry=zero2)
        def p2_cnt(c, carry):
            cnta, cntb = carry
            base = c * _L
            xv = xs[pl.ds(base, _L)]
            yv = ys[pl.ds(base, _L)]
            zv = zs[pl.ds(base, _L)]
            idxv = iota + base
            d2a = d2_from(xv, yv, zv, qa)
            hita = jnp.logical_and(d2a < ta, idxv != rva)
            plsc.store_compressed(hitk.at[pl.ds(cnta, _L)], d2a, mask=hita)
            plsc.store_compressed(hitv.at[pl.ds(cnta, _L)], idxv, mask=hita)
            d2b = d2_from(xv, yv, zv, qb)
            hitb = jnp.logical_and(d2b < tb, idxv != rvb)
            plsc.store_compressed(hitk2.at[pl.ds(cntb, _L)], d2b, mask=hitb)
            plsc.store_compressed(hitv2.at[pl.ds(cntb, _L)], idxv, mask=hitb)
            return (cnta + plsc.all_reduce_population_count(hita)[0],
                    cntb + plsc.all_reduce_population_count(hitb)[0])

        cnta, cntb = p2_cnt
        bka, bva = merge_hits(cnta, bka, bva, hitk, hitv)
        bkb, bvb = merge_hits(cntb, bkb, bvb, hitk2, hitv2)

        term = edge_loss(rva, bka, bva) + edge_loss(rvb, bkb, bvb)
        y = term - comp
        t = acc + y
        comp = (t - acc) - y
        return t, comp

    zero = jnp.zeros((_L,), jnp.float32)
    acc, _ = lax.fori_loop(0, _ROWS // 2, pair_body, (zero, zero))
    accv[...] = acc
    pltpu.sync_copy(accv, out_hbm.at[wid])


@jax.jit
def _partials(rx, ry, rz, px, py, pz):
    mesh = plsc.VectorSubcoreMesh(
        core_axis_name="c", subcore_axis_name="s",
        num_cores=_NC, num_subcores=_NS)
    f = pl.kernel(
        _body,
        out_type=jax.ShapeDtypeStruct((_NW, _L), jnp.float32),
        mesh=mesh,
        scratch_types=[
            pltpu.VMEM((_N,), jnp.float32),
            pltpu.VMEM((_N,), jnp.float32),
            pltpu.VMEM((_N,), jnp.float32),
            pltpu.VMEM((_N,), jnp.float32),
            pltpu.VMEM((_N,), jnp.float32),
            pltpu.VMEM((_N,), jnp.float32),
            pltpu.VMEM((_HCAP,), jnp.float32),
            pltpu.VMEM((_HCAP,), jnp.int32),
            pltpu.VMEM((_HCAP,), jnp.float32),
            pltpu.VMEM((_HCAP,), jnp.int32),
            pltpu.VMEM((_L,), jnp.float32),
        ],
        compiler_params=pltpu.CompilerParams(needs_layout_passes=False),
    )
    return f(rx, ry, rz, px, py, pz)


def kernel(points_ref, points):
    rx, ry, rz = (points_ref[:, :, i].reshape(-1) for i in range(3))
    px, py, pz = (points[:, :, i].reshape(-1) for i in range(3))
    partials = _partials(rx, ry, rz, px, py, pz)
    return jnp.sum(partials) / jnp.float32(_B * _N * _K)


# speedup vs baseline: 4.3754x; 1.3075x over previous
import functools

import numpy as np
import jax
import jax.numpy as jnp
from jax import lax
from jax.experimental import pallas as pl
from jax.experimental.pallas import tpu as pltpu
from jax.experimental.pallas import tpu_sc as plsc

_B = 4
_N = 4096
_K = 16
_L = 16
_NBLK = _N // _L
_NC = 2
_NS = 16
_NW = _NC * _NS
_WPB = _NW // _B
_ROWS = _N // _WPB
_BIG = np.float32(3.0e38)
_SAMPLE = 256
_HCAP = _N + _L


def _sqrt16(a):
    i = plsc.bitcast(a, jnp.int32)
    i = jnp.int32(0x5F3759DF) - (i >> 1)
    y = plsc.bitcast(i, jnp.float32)
    ah = a * jnp.float32(0.5)
    y = y * (jnp.float32(1.5) - ah * y * y)
    y = y * (jnp.float32(1.5) - ah * y * y)
    y = y * (jnp.float32(1.5) - ah * y * y)
    return jnp.where(a > 0.0, a * y, jnp.float32(0.0))


def _body(rx_hbm, ry_hbm, rz_hbm, px_hbm, py_hbm, pz_hbm, out_hbm,
          xs, ys, zs, pxs, pys, pzs, sqc, hitk, hitv, hitk2, hitv2, accv):
    wid = lax.axis_index("s") * _NC + lax.axis_index("c")
    batch = wid // _WPB
    row0 = (wid % _WPB) * _ROWS

    boff = batch * _N
    pltpu.sync_copy(rx_hbm.at[pl.ds(boff, _N)], xs)
    pltpu.sync_copy(ry_hbm.at[pl.ds(boff, _N)], ys)
    pltpu.sync_copy(rz_hbm.at[pl.ds(boff, _N)], zs)
    pltpu.sync_copy(px_hbm.at[pl.ds(boff, _N)], pxs)
    pltpu.sync_copy(py_hbm.at[pl.ds(boff, _N)], pys)
    pltpu.sync_copy(pz_hbm.at[pl.ds(boff, _N)], pzs)

    iota = lax.iota(jnp.int32, _L)

    @plsc.parallel_loop(0, _NBLK, unroll=4)
    def _sq(c):
        base = c * _L
        xv = xs[pl.ds(base, _L)]
        yv = ys[pl.ds(base, _L)]
        zv = zs[pl.ds(base, _L)]
        sqc[pl.ds(base, _L)] = xv * xv + yv * yv + zv * zv

    def key_block(base, q):
        xv = xs[pl.ds(base, _L)]
        yv = ys[pl.ds(base, _L)]
        zv = zs[pl.ds(base, _L)]
        sc = sqc[pl.ds(base, _L)]
        t0 = q[0] * xv + q[1] * yv + q[2] * zv
        return sc - 2.0 * t0

    def merge(carry, d2, idxv):
        bk, bv = carry
        sk, sv = plsc.sort_key_val(d2, idxv)
        rk = lax.rev(sk, (0,))
        rsv = lax.rev(sv, (0,))
        take = bk <= rk
        mk = jnp.where(take, bk, rk)
        mv = jnp.where(take, bv, rsv)
        nk, nv = plsc.sort_key_val(mk, mv)
        return nk, nv

    def sample_top16(q, rv):
        def p1_body(c, carry):
            base = c * _L
            v = key_block(base, q)
            idxv = iota + base
            v = jnp.where(idxv == rv, _BIG, v)
            return merge(carry, v, idxv)

        bk0 = jnp.full((_L,), _BIG, jnp.float32)
        bv0 = jnp.zeros((_L,), jnp.int32)
        return lax.fori_loop(0, _SAMPLE // _L, p1_body, (bk0, bv0))

    def merge_hits(cnt, bk, bv, hk_ref, hv_ref, rv):
        def p3_body(j, carry):
            base = j * _L
            hk = hk_ref[pl.ds(base, _L)]
            hv = hv_ref[pl.ds(base, _L)]
            hk = jnp.where(iota + base < cnt, hk, _BIG)
            hk = jnp.where(hv == rv, _BIG, hk)
            return merge(carry, hk, hv)

        nit = (cnt + _L - 1) // _L
        return lax.fori_loop(0, nit, p3_body, (bk, bv))

    def edge_loss(rv, bk, bv):
        sqq = plsc.load_gather(sqc, [rv])
        dist_ref = _sqrt16(bk + sqq)
        qpx = plsc.load_gather(pxs, [rv])
        qpy = plsc.load_gather(pys, [rv])
        qpz = plsc.load_gather(pzs, [rv])
        nx = plsc.load_gather(pxs, [bv])
        ny = plsc.load_gather(pys, [bv])
        nz = plsc.load_gather(pzs, [bv])
        ddx = nx - qpx
        ddy = ny - qpy
        ddz = nz - qpz
        dist = _sqrt16(ddx * ddx + ddy * ddy + ddz * ddz)
        return jnp.abs(dist_ref - dist)

    def pair_body(p, acc_comp):
        acc, comp = acc_comp
        ra = row0 + 2 * p
        rva = jnp.full((_L,), ra, jnp.int32)
        rvb = rva + 1
        qa = tuple(plsc.load_gather(s, [rva]) for s in (xs, ys, zs))
        qb = tuple(plsc.load_gather(s, [rvb]) for s in (xs, ys, zs))

        bka, bva = sample_top16(qa, rva)
        bkb, bvb = sample_top16(qb, rvb)
        ta = jnp.max(bka)
        tb = jnp.max(bkb)

        zero2 = (jnp.int32(0), jnp.int32(0))

        @plsc.parallel_loop(_SAMPLE // _L, _NBLK, unroll=4, carry=zero2)
        def p2_cnt(c, carry):
            cnta, cntb = carry
            base = c * _L
            xv = xs[pl.ds(base, _L)]
            yv = ys[pl.ds(base, _L)]
            zv = zs[pl.ds(base, _L)]
            sc = sqc[pl.ds(base, _L)]
            idxv = iota + base
            va = sc - 2.0 * (qa[0] * xv + qa[1] * yv + qa[2] * zv)
            hita = va < ta
            plsc.store_compressed(hitk.at[pl.ds(cnta, _L)], va, mask=hita)
            plsc.store_compressed(hitv.at[pl.ds(cnta, _L)], idxv, mask=hita)
            vb = sc - 2.0 * (qb[0] * xv + qb[1] * yv + qb[2] * zv)
            hitb = vb < tb
            plsc.store_compressed(hitk2.at[pl.ds(cntb, _L)], vb, mask=hitb)
            plsc.store_compressed(hitv2.at[pl.ds(cntb, _L)], idxv, mask=hitb)
            return (cnta + plsc.all_reduce_population_count(hita)[0],
                    cntb + plsc.all_reduce_population_count(hitb)[0])

        cnta, cntb = p2_cnt
        bka, bva = merge_hits(cnta, bka, bva, hitk, hitv, rva)
        bkb, bvb = merge_hits(cntb, bkb, bvb, hitk2, hitv2, rvb)

        term = edge_loss(rva, bka, bva) + edge_loss(rvb, bkb, bvb)
        y = term - comp
        t = acc + y
        comp = (t - acc) - y
        return t, comp

    zero = jnp.zeros((_L,), jnp.float32)
    acc, _ = lax.fori_loop(0, _ROWS // 2, pair_body, (zero, zero))
    accv[...] = acc
    pltpu.sync_copy(accv, out_hbm.at[wid])


@jax.jit
def _partials(rx, ry, rz, px, py, pz):
    mesh = plsc.VectorSubcoreMesh(
        core_axis_name="c", subcore_axis_name="s",
        num_cores=_NC, num_subcores=_NS)
    f = pl.kernel(
        _body,
        out_type=jax.ShapeDtypeStruct((_NW, _L), jnp.float32),
        mesh=mesh,
        scratch_types=[
            pltpu.VMEM((_N,), jnp.float32),
            pltpu.VMEM((_N,), jnp.float32),
            pltpu.VMEM((_N,), jnp.float32),
            pltpu.VMEM((_N,), jnp.float32),
            pltpu.VMEM((_N,), jnp.float32),
            pltpu.VMEM((_N,), jnp.float32),
            pltpu.VMEM((_N,), jnp.float32),
            pltpu.VMEM((_HCAP,), jnp.float32),
            pltpu.VMEM((_HCAP,), jnp.int32),
            pltpu.VMEM((_HCAP,), jnp.float32),
            pltpu.VMEM((_HCAP,), jnp.int32),
            pltpu.VMEM((_L,), jnp.float32),
        ],
        compiler_params=pltpu.CompilerParams(needs_layout_passes=False),
    )
    return f(rx, ry, rz, px, py, pz)


def kernel(points_ref, points):
    rx, ry, rz = (points_ref[:, :, i].reshape(-1) for i in range(3))
    px, py, pz = (points[:, :, i].reshape(-1) for i in range(3))
    partials = _partials(rx, ry, rz, px, py, pz)
    return jnp.sum(partials) / jnp.float32(_B * _N * _K)


# speedup vs baseline: 4.4047x; 1.0067x over previous
import functools

import numpy as np
import jax
import jax.numpy as jnp
from jax import lax
from jax.experimental import pallas as pl
from jax.experimental.pallas import tpu as pltpu
from jax.experimental.pallas import tpu_sc as plsc

_B = 4
_N = 4096
_K = 16
_L = 16
_NBLK = _N // _L
_NC = 2
_NS = 16
_NW = _NC * _NS
_WPB = _NW // _B
_ROWS = _N // _WPB
_BIG = np.float32(3.0e38)
_SAMPLE = 256
_HCAP = _N + _L


def _sqrt16(a):
    i = plsc.bitcast(a, jnp.int32)
    i = jnp.int32(0x5F3759DF) - (i >> 1)
    y = plsc.bitcast(i, jnp.float32)
    ah = a * jnp.float32(0.5)
    y = y * (jnp.float32(1.5) - ah * y * y)
    y = y * (jnp.float32(1.5) - ah * y * y)
    y = y * (jnp.float32(1.5) - ah * y * y)
    return jnp.where(a > 0.0, a * y, jnp.float32(0.0))


def _body(rx_hbm, ry_hbm, rz_hbm, px_hbm, py_hbm, pz_hbm, out_hbm,
          xs, ys, zs, pxs, pys, pzs, sqc, hitk, hitv, hitk2, hitv2,
          hitk3, hitv3, hitk4, hitv4, accv):
    wid = lax.axis_index("s") * _NC + lax.axis_index("c")
    batch = wid // _WPB
    row0 = (wid % _WPB) * _ROWS

    boff = batch * _N
    pltpu.sync_copy(rx_hbm.at[pl.ds(boff, _N)], xs)
    pltpu.sync_copy(ry_hbm.at[pl.ds(boff, _N)], ys)
    pltpu.sync_copy(rz_hbm.at[pl.ds(boff, _N)], zs)
    pltpu.sync_copy(px_hbm.at[pl.ds(boff, _N)], pxs)
    pltpu.sync_copy(py_hbm.at[pl.ds(boff, _N)], pys)
    pltpu.sync_copy(pz_hbm.at[pl.ds(boff, _N)], pzs)

    iota = lax.iota(jnp.int32, _L)

    @plsc.parallel_loop(0, _NBLK, unroll=4)
    def _sq(c):
        base = c * _L
        xv = xs[pl.ds(base, _L)]
        yv = ys[pl.ds(base, _L)]
        zv = zs[pl.ds(base, _L)]
        sqc[pl.ds(base, _L)] = xv * xv + yv * yv + zv * zv

    def key_block(base, q):
        xv = xs[pl.ds(base, _L)]
        yv = ys[pl.ds(base, _L)]
        zv = zs[pl.ds(base, _L)]
        sc = sqc[pl.ds(base, _L)]
        t0 = q[0] * xv + q[1] * yv + q[2] * zv
        return sc - 2.0 * t0

    def merge(carry, d2, idxv):
        bk, bv = carry
        sk, sv = plsc.sort_key_val(d2, idxv)
        rk = lax.rev(sk, (0,))
        rsv = lax.rev(sv, (0,))
        take = bk <= rk
        mk = jnp.where(take, bk, rk)
        mv = jnp.where(take, bv, rsv)
        nk, nv = plsc.sort_key_val(mk, mv)
        return nk, nv

    def sample_top16x4(qs, rvs):
        def p1_body(c, carry):
            base = c * _L
            xv = xs[pl.ds(base, _L)]
            yv = ys[pl.ds(base, _L)]
            zv = zs[pl.ds(base, _L)]
            sc = sqc[pl.ds(base, _L)]
            idxv = iota + base
            out = []
            for q, rv, ch in zip(qs, rvs, carry):
                v = sc - 2.0 * (q[0] * xv + q[1] * yv + q[2] * zv)
                v = jnp.where(idxv == rv, _BIG, v)
                out.append(merge(ch, v, idxv))
            return tuple(out)

        bk0 = jnp.full((_L,), _BIG, jnp.float32)
        bv0 = jnp.zeros((_L,), jnp.int32)
        init = tuple((bk0, bv0) for _ in range(4))
        return lax.fori_loop(0, _SAMPLE // _L, p1_body, init)

    def merge_hits(cnt, bk, bv, hk_ref, hv_ref, rv):
        def p3_body(j, carry):
            base = j * _L
            hk = hk_ref[pl.ds(base, _L)]
            hv = hv_ref[pl.ds(base, _L)]
            hk = jnp.where(iota + base < cnt, hk, _BIG)
            hk = jnp.where(hv == rv, _BIG, hk)
            return merge(carry, hk, hv)

        nit = (cnt + _L - 1) // _L
        return lax.fori_loop(0, nit, p3_body, (bk, bv))

    def edge_loss(rv, bk, bv):
        sqq = plsc.load_gather(sqc, [rv])
        dist_ref = _sqrt16(bk + sqq)
        qpx = plsc.load_gather(pxs, [rv])
        qpy = plsc.load_gather(pys, [rv])
        qpz = plsc.load_gather(pzs, [rv])
        nx = plsc.load_gather(pxs, [bv])
        ny = plsc.load_gather(pys, [bv])
        nz = plsc.load_gather(pzs, [bv])
        ddx = nx - qpx
        ddy = ny - qpy
        ddz = nz - qpz
        dist = _sqrt16(ddx * ddx + ddy * ddy + ddz * ddz)
        return jnp.abs(dist_ref - dist)

    def quad_body(p, acc_comp):
        acc, comp = acc_comp
        ra = row0 + 4 * p
        rva = jnp.full((_L,), ra, jnp.int32)
        rvs = (rva, rva + 1, rva + 2, rva + 3)
        qs = tuple(tuple(plsc.load_gather(s, [rv]) for s in (xs, ys, zs))
                   for rv in rvs)

        chains = sample_top16x4(qs, rvs)
        ts = tuple(jnp.max(ch[0]) for ch in chains)

        hrefs = ((hitk, hitv), (hitk2, hitv2), (hitk3, hitv3), (hitk4, hitv4))
        zero4 = (jnp.int32(0),) * 4

        @plsc.parallel_loop(_SAMPLE // _L, _NBLK, unroll=4, carry=zero4)
        def p2_cnt(c, cnts):
            base = c * _L
            xv = xs[pl.ds(base, _L)]
            yv = ys[pl.ds(base, _L)]
            zv = zs[pl.ds(base, _L)]
            sc = sqc[pl.ds(base, _L)]
            idxv = iota + base
            out = []
            for q, t, (hk_ref, hv_ref), cnt in zip(qs, ts, hrefs, cnts):
                v = sc - 2.0 * (q[0] * xv + q[1] * yv + q[2] * zv)
                hit = v < t
                plsc.store_compressed(hk_ref.at[pl.ds(cnt, _L)], v, mask=hit)
                plsc.store_compressed(hv_ref.at[pl.ds(cnt, _L)], idxv, mask=hit)
                out.append(cnt + plsc.all_reduce_population_count(hit)[0])
            return tuple(out)

        term = jnp.zeros((_L,), jnp.float32)
        for (bk, bv), cnt, (hk_ref, hv_ref), rv in zip(chains, p2_cnt, hrefs, rvs):
            bk, bv = merge_hits(cnt, bk, bv, hk_ref, hv_ref, rv)
            term = term + edge_loss(rv, bk, bv)

        y = term - comp
        t = acc + y
        comp = (t - acc) - y
        return t, comp

    zero = jnp.zeros((_L,), jnp.float32)
    acc, _ = lax.fori_loop(0, _ROWS // 4, quad_body, (zero, zero))
    accv[...] = acc
    pltpu.sync_copy(accv, out_hbm.at[wid])


@jax.jit
def _partials(rx, ry, rz, px, py, pz):
    mesh = plsc.VectorSubcoreMesh(
        core_axis_name="c", subcore_axis_name="s",
        num_cores=_NC, num_subcores=_NS)
    f = pl.kernel(
        _body,
        out_type=jax.ShapeDtypeStruct((_NW, _L), jnp.float32),
        mesh=mesh,
        scratch_types=[
            pltpu.VMEM((_N,), jnp.float32),
            pltpu.VMEM((_N,), jnp.float32),
            pltpu.VMEM((_N,), jnp.float32),
            pltpu.VMEM((_N,), jnp.float32),
            pltpu.VMEM((_N,), jnp.float32),
            pltpu.VMEM((_N,), jnp.float32),
            pltpu.VMEM((_N,), jnp.float32),
            pltpu.VMEM((_HCAP,), jnp.float32),
            pltpu.VMEM((_HCAP,), jnp.int32),
            pltpu.VMEM((_HCAP,), jnp.float32),
            pltpu.VMEM((_HCAP,), jnp.int32),
            pltpu.VMEM((_HCAP,), jnp.float32),
            pltpu.VMEM((_HCAP,), jnp.int32),
            pltpu.VMEM((_HCAP,), jnp.float32),
            pltpu.VMEM((_HCAP,), jnp.int32),
            pltpu.VMEM((_L,), jnp.float32),
        ],
        compiler_params=pltpu.CompilerParams(needs_layout_passes=False),
    )
    return f(rx, ry, rz, px, py, pz)


def kernel(points_ref, points):
    rx, ry, rz = (points_ref[:, :, i].reshape(-1) for i in range(3))
    px, py, pz = (points[:, :, i].reshape(-1) for i in range(3))
    partials = _partials(rx, ry, rz, px, py, pz)
    return jnp.sum(partials) / jnp.float32(_B * _N * _K)


# speedup vs baseline: 5.0635x; 1.1496x over previous
import functools

import numpy as np
import jax
import jax.numpy as jnp
from jax import lax
from jax.experimental import pallas as pl
from jax.experimental.pallas import tpu as pltpu
from jax.experimental.pallas import tpu_sc as plsc

_B = 4
_N = 4096
_K = 16
_L = 16
_NBLK = _N // _L
_NC = 2
_NS = 16
_NW = _NC * _NS
_WPB = _NW // _B
_ROWS = _N // _WPB
_BIG = np.float32(3.0e38)
_SAMPLE = 256
_HCAP = _N + _L


def _sqrt16(a):
    i = plsc.bitcast(a, jnp.int32)
    i = jnp.int32(0x5F3759DF) - (i >> 1)
    y = plsc.bitcast(i, jnp.float32)
    ah = a * jnp.float32(0.5)
    y = y * (jnp.float32(1.5) - ah * y * y)
    y = y * (jnp.float32(1.5) - ah * y * y)
    y = y * (jnp.float32(1.5) - ah * y * y)
    return jnp.where(a > 0.0, a * y, jnp.float32(0.0))


def _body(rx_hbm, ry_hbm, rz_hbm, px_hbm, py_hbm, pz_hbm, out_hbm,
          xs, ys, zs, pxs, pys, pzs, sqc, hitk, hitv, hitk2, hitv2,
          hitk3, hitv3, hitk4, hitv4, accv):
    wid = lax.axis_index("s") * _NC + lax.axis_index("c")
    batch = wid // _WPB
    row0 = (wid % _WPB) * _ROWS

    boff = batch * _N
    pltpu.sync_copy(rx_hbm.at[pl.ds(boff, _N)], xs)
    pltpu.sync_copy(ry_hbm.at[pl.ds(boff, _N)], ys)
    pltpu.sync_copy(rz_hbm.at[pl.ds(boff, _N)], zs)
    pltpu.sync_copy(px_hbm.at[pl.ds(boff, _N)], pxs)
    pltpu.sync_copy(py_hbm.at[pl.ds(boff, _N)], pys)
    pltpu.sync_copy(pz_hbm.at[pl.ds(boff, _N)], pzs)

    iota = lax.iota(jnp.int32, _L)

    @plsc.parallel_loop(0, _NBLK, unroll=4)
    def _sq(c):
        base = c * _L
        xv = xs[pl.ds(base, _L)]
        yv = ys[pl.ds(base, _L)]
        zv = zs[pl.ds(base, _L)]
        sqc[pl.ds(base, _L)] = xv * xv + yv * yv + zv * zv

    def key_block(base, q):
        xv = xs[pl.ds(base, _L)]
        yv = ys[pl.ds(base, _L)]
        zv = zs[pl.ds(base, _L)]
        sc = sqc[pl.ds(base, _L)]
        t0 = q[0] * xv + q[1] * yv + q[2] * zv
        return sc - 2.0 * t0

    def merge(carry, d2, idxv):
        bk, bv = carry
        sk, sv = plsc.sort_key_val(d2, idxv, descending=True)
        take = bk <= sk
        mk = jnp.where(take, bk, sk)
        mv = jnp.where(take, bv, sv)
        nk, nv = plsc.sort_key_val(mk, mv)
        return nk, nv

    def sample_top16x4(qs, rvs):
        def p1_body(c, carry):
            base = c * _L
            xv = xs[pl.ds(base, _L)]
            yv = ys[pl.ds(base, _L)]
            zv = zs[pl.ds(base, _L)]
            sc = sqc[pl.ds(base, _L)]
            idxv = iota + base
            out = []
            for q, rv, ch in zip(qs, rvs, carry):
                v = sc - 2.0 * (q[0] * xv + q[1] * yv + q[2] * zv)
                v = jnp.where(idxv == rv, _BIG, v)
                out.append(merge(ch, v, idxv))
            return tuple(out)

        bk0 = jnp.full((_L,), _BIG, jnp.float32)
        bv0 = jnp.zeros((_L,), jnp.int32)
        init = tuple((bk0, bv0) for _ in range(4))
        return lax.fori_loop(0, _SAMPLE // _L, p1_body, init)

    def merge_hits_x4(chains, cnts, hrefs, rvs):
        def p3_body(j, carry):
            base = j * _L
            out = []
            for (bk, bv), cnt, (hk_ref, hv_ref), rv in zip(
                    carry, cnts, hrefs, rvs):
                hk = hk_ref[pl.ds(base, _L)]
                hv = hv_ref[pl.ds(base, _L)]
                hk = jnp.where(iota + base < cnt, hk, _BIG)
                hk = jnp.where(hv == rv, _BIG, hk)
                out.append(merge((bk, bv), hk, hv))
            return tuple(out)

        cmax = jnp.maximum(jnp.maximum(cnts[0], cnts[1]),
                           jnp.maximum(cnts[2], cnts[3]))
        nit = (cmax + _L - 1) // _L
        return lax.fori_loop(0, nit, p3_body, chains)

    def edge_loss(rv, bk, bv):
        sqq = plsc.load_gather(sqc, [rv])
        dist_ref = _sqrt16(bk + sqq)
        qpx = plsc.load_gather(pxs, [rv])
        qpy = plsc.load_gather(pys, [rv])
        qpz = plsc.load_gather(pzs, [rv])
        nx = plsc.load_gather(pxs, [bv])
        ny = plsc.load_gather(pys, [bv])
        nz = plsc.load_gather(pzs, [bv])
        ddx = nx - qpx
        ddy = ny - qpy
        ddz = nz - qpz
        dist = _sqrt16(ddx * ddx + ddy * ddy + ddz * ddz)
        return jnp.abs(dist_ref - dist)

    def quad_body(p, acc_comp):
        acc, comp = acc_comp
        ra = row0 + 4 * p
        rva = jnp.full((_L,), ra, jnp.int32)
        rvs = (rva, rva + 1, rva + 2, rva + 3)
        qs = tuple(tuple(plsc.load_gather(s, [rv]) for s in (xs, ys, zs))
                   for rv in rvs)

        chains = sample_top16x4(qs, rvs)
        ts = tuple(jnp.max(ch[0]) for ch in chains)

        hrefs = ((hitk, hitv), (hitk2, hitv2), (hitk3, hitv3), (hitk4, hitv4))
        zero4 = (jnp.int32(0),) * 4

        @plsc.parallel_loop(_SAMPLE // _L, _NBLK, unroll=4, carry=zero4)
        def p2_cnt(c, cnts):
            base = c * _L
            xv = xs[pl.ds(base, _L)]
            yv = ys[pl.ds(base, _L)]
            zv = zs[pl.ds(base, _L)]
            sc = sqc[pl.ds(base, _L)]
            idxv = iota + base
            out = []
            for q, t, (hk_ref, hv_ref), cnt in zip(qs, ts, hrefs, cnts):
                v = sc - 2.0 * (q[0] * xv + q[1] * yv + q[2] * zv)
                hit = v < t
                plsc.store_compressed(hk_ref.at[pl.ds(cnt, _L)], v, mask=hit)
                plsc.store_compressed(hv_ref.at[pl.ds(cnt, _L)], idxv, mask=hit)
                out.append(cnt + plsc.all_reduce_population_count(hit)[0])
            return tuple(out)

        chains = merge_hits_x4(chains, p2_cnt, hrefs, rvs)
        term = jnp.zeros((_L,), jnp.float32)
        for (bk, bv), rv in zip(chains, rvs):
            term = term + edge_loss(rv, bk, bv)

        y = term - comp
        t = acc + y
        comp = (t - acc) - y
        return t, comp

    zero = jnp.zeros((_L,), jnp.float32)
    acc, _ = lax.fori_loop(0, _ROWS // 4, quad_body, (zero, zero))
    accv[...] = acc
    pltpu.sync_copy(accv, out_hbm.at[wid])


@jax.jit
def _partials(rx, ry, rz, px, py, pz):
    mesh = plsc.VectorSubcoreMesh(
        core_axis_name="c", subcore_axis_name="s",
        num_cores=_NC, num_subcores=_NS)
    f = pl.kernel(
        _body,
        out_type=jax.ShapeDtypeStruct((_NW, _L), jnp.float32),
        mesh=mesh,
        scratch_types=[
            pltpu.VMEM((_N,), jnp.float32),
            pltpu.VMEM((_N,), jnp.float32),
            pltpu.VMEM((_N,), jnp.float32),
            pltpu.VMEM((_N,), jnp.float32),
            pltpu.VMEM((_N,), jnp.float32),
            pltpu.VMEM((_N,), jnp.float32),
            pltpu.VMEM((_N,), jnp.float32),
            pltpu.VMEM((_HCAP,), jnp.float32),
            pltpu.VMEM((_HCAP,), jnp.int32),
            pltpu.VMEM((_HCAP,), jnp.float32),
            pltpu.VMEM((_HCAP,), jnp.int32),
            pltpu.VMEM((_HCAP,), jnp.float32),
            pltpu.VMEM((_HCAP,), jnp.int32),
            pltpu.VMEM((_HCAP,), jnp.float32),
            pltpu.VMEM((_HCAP,), jnp.int32),
            pltpu.VMEM((_L,), jnp.float32),
        ],
        compiler_params=pltpu.CompilerParams(needs_layout_passes=False),
    )
    return f(rx, ry, rz, px, py, pz)


def kernel(points_ref, points):
    rx, ry, rz = (points_ref[:, :, i].reshape(-1) for i in range(3))
    px, py, pz = (points[:, :, i].reshape(-1) for i in range(3))
    partials = _partials(rx, ry, rz, px, py, pz)
    return jnp.sum(partials) / jnp.float32(_B * _N * _K)


# speedup vs baseline: 5.1688x; 1.0208x over previous
import functools

import numpy as np
import jax
import jax.numpy as jnp
from jax import lax
from jax.experimental import pallas as pl
from jax.experimental.pallas import tpu as pltpu
from jax.experimental.pallas import tpu_sc as plsc

_B = 4
_N = 4096
_K = 16
_L = 16
_NBLK = _N // _L
_NC = 2
_NS = 16
_NW = _NC * _NS
_WPB = _NW // _B
_ROWS = _N // _WPB
_BIG = np.float32(3.0e38)
_SAMPLE = 256
_HCAP = _N + _L


def _sqrt16(a):
    i = plsc.bitcast(a, jnp.int32)
    i = jnp.int32(0x5F3759DF) - (i >> 1)
    y = plsc.bitcast(i, jnp.float32)
    ah = a * jnp.float32(0.5)
    y = y * (jnp.float32(1.5) - ah * y * y)
    y = y * (jnp.float32(1.5) - ah * y * y)
    y = y * (jnp.float32(1.5) - ah * y * y)
    return jnp.where(a > 0.0, a * y, jnp.float32(0.0))


def _body(rx_hbm, ry_hbm, rz_hbm, px_hbm, py_hbm, pz_hbm, out_hbm,
          xs, ys, zs, pxs, pys, pzs, sqc, hitv, hitv2, hitv3, hitv4, accv):
    wid = lax.axis_index("s") * _NC + lax.axis_index("c")
    batch = wid // _WPB
    row0 = (wid % _WPB) * _ROWS

    boff = batch * _N
    pltpu.sync_copy(rx_hbm.at[pl.ds(boff, _N)], xs)
    pltpu.sync_copy(ry_hbm.at[pl.ds(boff, _N)], ys)
    pltpu.sync_copy(rz_hbm.at[pl.ds(boff, _N)], zs)
    pltpu.sync_copy(px_hbm.at[pl.ds(boff, _N)], pxs)
    pltpu.sync_copy(py_hbm.at[pl.ds(boff, _N)], pys)
    pltpu.sync_copy(pz_hbm.at[pl.ds(boff, _N)], pzs)

    iota = lax.iota(jnp.int32, _L)

    @plsc.parallel_loop(0, _NBLK, unroll=4)
    def _sq(c):
        base = c * _L
        xv = xs[pl.ds(base, _L)]
        yv = ys[pl.ds(base, _L)]
        zv = zs[pl.ds(base, _L)]
        sqc[pl.ds(base, _L)] = xv * xv + yv * yv + zv * zv

    def key_block(base, q):
        xv = xs[pl.ds(base, _L)]
        yv = ys[pl.ds(base, _L)]
        zv = zs[pl.ds(base, _L)]
        sc = sqc[pl.ds(base, _L)]
        t0 = q[0] * xv + q[1] * yv + q[2] * zv
        return sc - 2.0 * t0

    def merge(carry, d2, idxv):
        bk, bv = carry
        sk, sv = plsc.sort_key_val(d2, idxv, descending=True)
        take = bk <= sk
        mk = jnp.where(take, bk, sk)
        mv = jnp.where(take, bv, sv)
        nk, nv = plsc.sort_key_val(mk, mv)
        return nk, nv

    def sample_top16x4(qs, rvs):
        def p1_body(c, carry):
            base = c * _L
            xv = xs[pl.ds(base, _L)]
            yv = ys[pl.ds(base, _L)]
            zv = zs[pl.ds(base, _L)]
            sc = sqc[pl.ds(base, _L)]
            idxv = iota + base
            out = []
            for q, rv, ch in zip(qs, rvs, carry):
                v = sc - 2.0 * (q[0] * xv + q[1] * yv + q[2] * zv)
                v = jnp.where(idxv == rv, _BIG, v)
                out.append(merge(ch, v, idxv))
            return tuple(out)

        bk0 = jnp.full((_L,), _BIG, jnp.float32)
        bv0 = jnp.zeros((_L,), jnp.int32)
        init = tuple((bk0, bv0) for _ in range(4))
        return lax.fori_loop(0, _SAMPLE // _L, p1_body, init)

    def merge_hits_x4(chains, cnts, hrefs, rvs, qs):
        def p3_body(j, carry):
            base = j * _L
            out = []
            for (bk, bv), cnt, hv_ref, rv, q in zip(
                    carry, cnts, hrefs, rvs, qs):
                valid = iota + base < cnt
                hv = hv_ref[pl.ds(base, _L)]
                cx = plsc.load_gather(xs, [hv], mask=valid)
                cy = plsc.load_gather(ys, [hv], mask=valid)
                cz = plsc.load_gather(zs, [hv], mask=valid)
                scv = plsc.load_gather(sqc, [hv], mask=valid)
                hk = scv - 2.0 * (q[0] * cx + q[1] * cy + q[2] * cz)
                hk = jnp.where(valid, hk, _BIG)
                hk = jnp.where(hv == rv, _BIG, hk)
                out.append(merge((bk, bv), hk, hv))
            return tuple(out)

        cmax = jnp.maximum(jnp.maximum(cnts[0], cnts[1]),
                           jnp.maximum(cnts[2], cnts[3]))
        nit = (cmax + _L - 1) // _L
        return lax.fori_loop(0, nit, p3_body, chains)

    def edge_loss(rv, bk, bv):
        sqq = plsc.load_gather(sqc, [rv])
        dist_ref = _sqrt16(bk + sqq)
        qpx = plsc.load_gather(pxs, [rv])
        qpy = plsc.load_gather(pys, [rv])
        qpz = plsc.load_gather(pzs, [rv])
        nx = plsc.load_gather(pxs, [bv])
        ny = plsc.load_gather(pys, [bv])
        nz = plsc.load_gather(pzs, [bv])
        ddx = nx - qpx
        ddy = ny - qpy
        ddz = nz - qpz
        dist = _sqrt16(ddx * ddx + ddy * ddy + ddz * ddz)
        return jnp.abs(dist_ref - dist)

    def quad_body(p, acc_comp):
        acc, comp = acc_comp
        ra = row0 + 4 * p
        rva = jnp.full((_L,), ra, jnp.int32)
        rvs = (rva, rva + 1, rva + 2, rva + 3)
        qs = tuple(tuple(plsc.load_gather(s, [rv]) for s in (xs, ys, zs))
                   for rv in rvs)

        chains = sample_top16x4(qs, rvs)
        ts = tuple(jnp.max(ch[0]) for ch in chains)

        hrefs = (hitv, hitv2, hitv3, hitv4)
        zero4 = (jnp.int32(0),) * 4

        @plsc.parallel_loop(_SAMPLE // _L, _NBLK, unroll=4, carry=zero4)
        def p2_cnt(c, cnts):
            base = c * _L
            xv = xs[pl.ds(base, _L)]
            yv = ys[pl.ds(base, _L)]
            zv = zs[pl.ds(base, _L)]
            sc = sqc[pl.ds(base, _L)]
            idxv = iota + base
            out = []
            for q, t, hv_ref, cnt in zip(qs, ts, hrefs, cnts):
                v = sc - 2.0 * (q[0] * xv + q[1] * yv + q[2] * zv)
                hit = v < t
                plsc.store_compressed(hv_ref.at[pl.ds(cnt, _L)], idxv, mask=hit)
                out.append(cnt + plsc.all_reduce_population_count(hit)[0])
            return tuple(out)

        chains = merge_hits_x4(chains, p2_cnt, hrefs, rvs, qs)
        term = jnp.zeros((_L,), jnp.float32)
        for (bk, bv), rv in zip(chains, rvs):
            term = term + edge_loss(rv, bk, bv)

        y = term - comp
        t = acc + y
        comp = (t - acc) - y
        return t, comp

    zero = jnp.zeros((_L,), jnp.float32)
    acc, _ = lax.fori_loop(0, _ROWS // 4, quad_body, (zero, zero))
    accv[...] = acc
    pltpu.sync_copy(accv, out_hbm.at[wid])


@jax.jit
def _partials(rx, ry, rz, px, py, pz):
    mesh = plsc.VectorSubcoreMesh(
        core_axis_name="c", subcore_axis_name="s",
        num_cores=_NC, num_subcores=_NS)
    f = pl.kernel(
        _body,
        out_type=jax.ShapeDtypeStruct((_NW, _L), jnp.float32),
        mesh=mesh,
        scratch_types=[
            pltpu.VMEM((_N,), jnp.float32),
            pltpu.VMEM((_N,), jnp.float32),
            pltpu.VMEM((_N,), jnp.float32),
            pltpu.VMEM((_N,), jnp.float32),
            pltpu.VMEM((_N,), jnp.float32),
            pltpu.VMEM((_N,), jnp.float32),
            pltpu.VMEM((_N,), jnp.float32),
            pltpu.VMEM((_HCAP,), jnp.int32),
            pltpu.VMEM((_HCAP,), jnp.int32),
            pltpu.VMEM((_HCAP,), jnp.int32),
            pltpu.VMEM((_HCAP,), jnp.int32),
            pltpu.VMEM((_L,), jnp.float32),
        ],
        compiler_params=pltpu.CompilerParams(needs_layout_passes=False),
    )
    return f(rx, ry, rz, px, py, pz)


def kernel(points_ref, points):
    rx, ry, rz = (points_ref[:, :, i].reshape(-1) for i in range(3))
    px, py, pz = (points[:, :, i].reshape(-1) for i in range(3))
    partials = _partials(rx, ry, rz, px, py, pz)
    return jnp.sum(partials) / jnp.float32(_B * _N * _K)
